# Initial kernel scaffold; baseline (speedup 1.0000x reference)
import jax
import jax.numpy as jnp
from jax.experimental import pallas as pl


def kernel(interaction, testId, assessmentItemID, knowledgeTag, emb_interaction, emb_testId, emb_assessmentItemID, emb_knowledgeTag):
    raise NotImplementedError("write your pallas kernel here")


# speedup vs baseline: 2.3578x; 2.3578x over previous
import functools
import jax
import jax.numpy as jnp
from jax import lax
from jax.experimental import pallas as pl
from jax.experimental.pallas import tpu as pltpu
from jax.experimental.pallas import tpu_sc as plsc

B, L = 4096, 200
N = B * L
D = 16
F = 4
NW = 32
PER_W = N // NW
CHUNK = 512
GRP = 128
N_CHUNKS = PER_W // CHUNK
N_GRP = CHUNK // GRP


def _emb_kernel(i0_hbm, i1_hbm, i2_hbm, i3_hbm,
                t0_hbm, t1_hbm, t2_hbm, t3_hbm,
                out_hbm,
                idx_v, rows_v, sem):
    wid = lax.axis_index("s") * 2 + lax.axis_index("c")
    idx_refs = (i0_hbm, i1_hbm, i2_hbm, i3_hbm)
    tbl_refs = (t0_hbm, t1_hbm, t2_hbm, t3_hbm)

    def chunk_body(k, carry):
        base = wid * PER_W + k * CHUNK
        for f in range(F):
            pltpu.sync_copy(idx_refs[f].at[pl.ds(base, CHUNK)],
                            idx_v.at[f])
        copies = []
        for f in range(F):
            for g in range(N_GRP):
                copies.append(pltpu.async_copy(
                    tbl_refs[f].at[idx_v.at[f, pl.ds(g * GRP, GRP)]],
                    rows_v.at[f, pl.ds(g * GRP, GRP)],
                    sem))
        for c in copies:
            c.wait()
        for f in range(F):
            pltpu.sync_copy(rows_v.at[f], out_hbm.at[pl.ds(base, CHUNK), f])
        return carry

    lax.fori_loop(0, N_CHUNKS, chunk_body, 0)


@jax.jit
def _run(i0, i1, i2, i3, t0, t1, t2, t3):
    mesh = plsc.VectorSubcoreMesh(core_axis_name="c", subcore_axis_name="s")
    k = functools.partial(
        pl.kernel,
        out_type=jax.ShapeDtypeStruct((N, F, D), jnp.float32),
        mesh=mesh,
        scratch_types=[
            pltpu.VMEM((F, CHUNK), jnp.int32),
            pltpu.VMEM((F, CHUNK, D), jnp.float32),
            pltpu.SemaphoreType.DMA,
        ],
        compiler_params=pltpu.CompilerParams(use_tc_tiling_on_sc=False),
    )(_emb_kernel)
    return k(i0, i1, i2, i3, t0, t1, t2, t3)


def kernel(interaction, testId, assessmentItemID, knowledgeTag,
           emb_interaction, emb_testId, emb_assessmentItemID,
           emb_knowledgeTag):
    i0 = interaction.astype(jnp.int32).reshape(N)
    i1 = testId.astype(jnp.int32).reshape(N)
    i2 = assessmentItemID.astype(jnp.int32).reshape(N)
    i3 = knowledgeTag.astype(jnp.int32).reshape(N)
    out = _run(i0, i1, i2, i3,
               emb_interaction, emb_testId, emb_assessmentItemID,
               emb_knowledgeTag)
    return out.reshape(B, L, F * D)
